# Initial kernel scaffold; baseline (speedup 1.0000x reference)
import jax
import jax.numpy as jnp
from jax.experimental import pallas as pl


def kernel(x, positions, W, a_src, a_dst, ln_scale, ln_bias, topk):
    raise NotImplementedError("write your pallas kernel here")


# speedup vs baseline: 14.0470x; 14.0470x over previous
import jax
import jax.numpy as jnp
from jax.experimental import pallas as pl

B, N, IN_F = 2, 4096, 128
H, D = 4, 32
K = 32
R = 256


def _prologue_kernel(x_ref, w_ref, asel_ref, eind_ref, hwe_ref):
    x = x_ref[0]
    W = w_ref[...]
    h = jnp.dot(x.astype(jnp.bfloat16), W.astype(jnp.bfloat16),
                preferred_element_type=jnp.float32)
    e = jnp.dot(h, asel_ref[...], precision=jax.lax.Precision.HIGHEST)
    f = jnp.where(e >= 0.0, e, 0.2 * e)
    fmax = jnp.max(f, axis=0, keepdims=True)
    expf = jnp.exp(f - fmax)
    expfb = jnp.dot(expf, eind_ref[...],
                    precision=jax.lax.Precision.HIGHEST)
    hwe_ref[0, :, : H * D] = h * expfb
    hwe_ref[0, :, H * D:] = expfb


def _main_kernel(pos_ref, post_ref, hwe_ref, x_ref, lns_ref, lnb_ref, out_ref):
    pos = pos_ref[0]
    posT = post_ref[0]
    dot = jnp.dot(pos.astype(jnp.bfloat16), posT.astype(jnp.bfloat16),
                  preferred_element_type=jnp.float32)
    p2b = jnp.sum(pos * pos, axis=1, keepdims=True)
    p2f = jnp.sum(posT * posT, axis=0, keepdims=True)
    d2 = p2b + p2f - 2.0 * dot

    def body(_, d2w):
        m = jnp.min(d2w, axis=1, keepdims=True)
        return jnp.where(d2w == m, jnp.inf, d2w)

    d2w = jax.lax.fori_loop(0, K, body, d2)
    A = jnp.isinf(d2w).astype(jnp.float32)

    nm = jnp.dot(A, hwe_ref[0], precision=jax.lax.Precision.HIGHEST)
    hp = nm[:, : H * D] / nm[:, H * D:]
    y = hp + x_ref[0]
    mu = jnp.mean(y, axis=1, keepdims=True)
    yc = y - mu
    var = jnp.mean(yc * yc, axis=1, keepdims=True)
    out = yc * jax.lax.rsqrt(var + 1e-5) * lns_ref[...] + lnb_ref[...]
    out_ref[0] = out


def kernel(x, positions, W, a_src, a_dst, ln_scale, ln_bias, topk):
    HD = H * D
    a_flat = (a_src + a_dst).reshape(HD)
    grp = jnp.arange(HD, dtype=jnp.int32) // D
    heads = jnp.arange(H, dtype=jnp.int32)
    asel = jnp.where(grp[:, None] == heads[None, :], a_flat[:, None], 0.0)
    eind = (grp[None, :] == heads[:, None]).astype(jnp.float32)
    posT = jnp.transpose(positions, (0, 2, 1))
    lns = ln_scale.reshape(1, HD)
    lnb = ln_bias.reshape(1, HD)

    hwe = pl.pallas_call(
        _prologue_kernel,
        grid=(B,),
        in_specs=[
            pl.BlockSpec((1, N, IN_F), lambda b: (b, 0, 0)),
            pl.BlockSpec((IN_F, HD), lambda b: (0, 0)),
            pl.BlockSpec((HD, H), lambda b: (0, 0)),
            pl.BlockSpec((H, HD), lambda b: (0, 0)),
        ],
        out_specs=pl.BlockSpec((1, N, 2 * HD), lambda b: (b, 0, 0)),
        out_shape=jax.ShapeDtypeStruct((B, N, 2 * HD), jnp.float32),
    )(x, W, asel, eind)

    out = pl.pallas_call(
        _main_kernel,
        grid=(B, N // R),
        in_specs=[
            pl.BlockSpec((1, R, 3), lambda b, i: (b, i, 0)),
            pl.BlockSpec((1, 3, N), lambda b, i: (b, 0, 0)),
            pl.BlockSpec((1, N, 2 * HD), lambda b, i: (b, 0, 0)),
            pl.BlockSpec((1, R, IN_F), lambda b, i: (b, i, 0)),
            pl.BlockSpec((1, HD), lambda b, i: (0, 0)),
            pl.BlockSpec((1, HD), lambda b, i: (0, 0)),
        ],
        out_specs=pl.BlockSpec((1, R, IN_F), lambda b, i: (b, i, 0)),
        out_shape=jax.ShapeDtypeStruct((B, N, IN_F), jnp.float32),
    )(positions, posT, hwe, x, lns, lnb)
    return out


# speedup vs baseline: 14.1187x; 1.0051x over previous
import jax
import jax.numpy as jnp
from jax.experimental import pallas as pl
from jax.experimental.pallas import tpu as pltpu

B, N, IN_F = 2, 4096, 128
H, D = 4, 32
K = 32
R = 256


def _prologue_kernel(x_ref, w_ref, asel_ref, eind_ref, hwe_ref):
    x = x_ref[0]
    W = w_ref[...]
    h = jnp.dot(x.astype(jnp.bfloat16), W.astype(jnp.bfloat16),
                preferred_element_type=jnp.float32)
    e = jnp.dot(h, asel_ref[...], precision=jax.lax.Precision.HIGHEST)
    f = jnp.where(e >= 0.0, e, 0.2 * e)
    fmax = jnp.max(f, axis=0, keepdims=True)
    expf = jnp.exp(f - fmax)
    expfb = jnp.dot(expf, eind_ref[...],
                    precision=jax.lax.Precision.HIGHEST)
    hwe_ref[0, :, : H * D] = h * expfb
    hwe_ref[0, :, H * D:] = expfb


def _main_kernel(pos_ref, post_ref, hwe_ref, x_ref, lns_ref, lnb_ref, out_ref):
    pos = pos_ref[0]
    posT = post_ref[0]
    dot = jnp.dot(pos.astype(jnp.bfloat16), posT.astype(jnp.bfloat16),
                  preferred_element_type=jnp.float32)
    p2b = jnp.sum(pos * pos, axis=1, keepdims=True)
    p2f = jnp.sum(posT * posT, axis=0, keepdims=True)
    d2 = p2b + p2f - 2.0 * dot

    def body(_, d2w):
        m = jnp.min(d2w, axis=1, keepdims=True)
        return jnp.where(d2w == m, jnp.inf, d2w)

    d2w = jax.lax.fori_loop(0, K, body, d2)
    A = jnp.isinf(d2w).astype(jnp.float32)

    nm = jnp.dot(A, hwe_ref[0], precision=jax.lax.Precision.HIGHEST)
    hp = nm[:, : H * D] / nm[:, H * D:]
    y = hp + x_ref[0]
    mu = jnp.mean(y, axis=1, keepdims=True)
    yc = y - mu
    var = jnp.mean(yc * yc, axis=1, keepdims=True)
    out = yc * jax.lax.rsqrt(var + 1e-5) * lns_ref[...] + lnb_ref[...]
    out_ref[0] = out


def kernel(x, positions, W, a_src, a_dst, ln_scale, ln_bias, topk):
    HD = H * D
    a_flat = (a_src + a_dst).reshape(HD)
    grp = jnp.arange(HD, dtype=jnp.int32) // D
    heads = jnp.arange(H, dtype=jnp.int32)
    asel = jnp.where(grp[:, None] == heads[None, :], a_flat[:, None], 0.0)
    eind = (grp[None, :] == heads[:, None]).astype(jnp.float32)
    posT = jnp.transpose(positions, (0, 2, 1))
    lns = ln_scale.reshape(1, HD)
    lnb = ln_bias.reshape(1, HD)

    hwe = pl.pallas_call(
        _prologue_kernel,
        grid=(B,),
        in_specs=[
            pl.BlockSpec((1, N, IN_F), lambda b: (b, 0, 0)),
            pl.BlockSpec((IN_F, HD), lambda b: (0, 0)),
            pl.BlockSpec((HD, H), lambda b: (0, 0)),
            pl.BlockSpec((H, HD), lambda b: (0, 0)),
        ],
        out_specs=pl.BlockSpec((1, N, 2 * HD), lambda b: (b, 0, 0)),
        out_shape=jax.ShapeDtypeStruct((B, N, 2 * HD), jnp.float32),
        compiler_params=pltpu.CompilerParams(
            dimension_semantics=("parallel",)),
    )(x, W, asel, eind)

    out = pl.pallas_call(
        _main_kernel,
        grid=(B, N // R),
        in_specs=[
            pl.BlockSpec((1, R, 3), lambda b, i: (b, i, 0)),
            pl.BlockSpec((1, 3, N), lambda b, i: (b, 0, 0)),
            pl.BlockSpec((1, N, 2 * HD), lambda b, i: (b, 0, 0)),
            pl.BlockSpec((1, R, IN_F), lambda b, i: (b, i, 0)),
            pl.BlockSpec((1, HD), lambda b, i: (0, 0)),
            pl.BlockSpec((1, HD), lambda b, i: (0, 0)),
        ],
        out_specs=pl.BlockSpec((1, R, IN_F), lambda b, i: (b, i, 0)),
        out_shape=jax.ShapeDtypeStruct((B, N, IN_F), jnp.float32),
        compiler_params=pltpu.CompilerParams(
            dimension_semantics=("parallel", "parallel")),
    )(positions, posT, hwe, x, lns, lnb)
    return out


# speedup vs baseline: 27.7436x; 1.9650x over previous
import jax
import jax.numpy as jnp
from jax.experimental import pallas as pl
from jax.experimental.pallas import tpu as pltpu

B, N, IN_F = 2, 4096, 128
H, D = 4, 32
K = 32
R = 256


def _prologue_kernel(x_ref, w_ref, asel_ref, eind_ref, hwe_ref):
    x = x_ref[0]
    W = w_ref[...]
    h = jnp.dot(x.astype(jnp.bfloat16), W.astype(jnp.bfloat16),
                preferred_element_type=jnp.float32)
    e = jnp.dot(h, asel_ref[...], precision=jax.lax.Precision.HIGHEST)
    f = jnp.where(e >= 0.0, e, 0.2 * e)
    fmax = jnp.max(f, axis=0, keepdims=True)
    expf = jnp.exp(f - fmax)
    expfb = jnp.dot(expf, eind_ref[...],
                    precision=jax.lax.Precision.HIGHEST)
    hwe_ref[0, :, : H * D] = h * expfb
    hwe_ref[0, :, H * D:] = expfb


def _main_kernel(pos_ref, post_ref, hwe_ref, x_ref, lns_ref, lnb_ref, out_ref):
    pos = pos_ref[0]
    posT = post_ref[0]
    dot = jnp.dot(pos.astype(jnp.bfloat16), posT.astype(jnp.bfloat16),
                  preferred_element_type=jnp.float32)
    p2b = jnp.sum(pos * pos, axis=1, keepdims=True)
    p2f = jnp.sum(posT * posT, axis=0, keepdims=True)
    d2 = p2b + p2f - 2.0 * dot

    def body(_, m):
        return jnp.min(jnp.where(d2 > m, d2, jnp.inf), axis=1, keepdims=True)

    m0 = jnp.full((R, 1), -jnp.inf, dtype=jnp.float32)
    tau = jax.lax.fori_loop(0, K, body, m0)
    A = (d2 <= tau).astype(jnp.float32)

    nm = jnp.dot(A, hwe_ref[0], precision=jax.lax.Precision.HIGHEST)
    hp = nm[:, : H * D] / nm[:, H * D:]
    y = hp + x_ref[0]
    mu = jnp.mean(y, axis=1, keepdims=True)
    yc = y - mu
    var = jnp.mean(yc * yc, axis=1, keepdims=True)
    out = yc * jax.lax.rsqrt(var + 1e-5) * lns_ref[...] + lnb_ref[...]
    out_ref[0] = out


def kernel(x, positions, W, a_src, a_dst, ln_scale, ln_bias, topk):
    HD = H * D
    a_flat = (a_src + a_dst).reshape(HD)
    grp = jnp.arange(HD, dtype=jnp.int32) // D
    heads = jnp.arange(H, dtype=jnp.int32)
    asel = jnp.where(grp[:, None] == heads[None, :], a_flat[:, None], 0.0)
    eind = (grp[None, :] == heads[:, None]).astype(jnp.float32)
    posT = jnp.transpose(positions, (0, 2, 1))
    lns = ln_scale.reshape(1, HD)
    lnb = ln_bias.reshape(1, HD)

    hwe = pl.pallas_call(
        _prologue_kernel,
        grid=(B,),
        in_specs=[
            pl.BlockSpec((1, N, IN_F), lambda b: (b, 0, 0)),
            pl.BlockSpec((IN_F, HD), lambda b: (0, 0)),
            pl.BlockSpec((HD, H), lambda b: (0, 0)),
            pl.BlockSpec((H, HD), lambda b: (0, 0)),
        ],
        out_specs=pl.BlockSpec((1, N, 2 * HD), lambda b: (b, 0, 0)),
        out_shape=jax.ShapeDtypeStruct((B, N, 2 * HD), jnp.float32),
        compiler_params=pltpu.CompilerParams(
            dimension_semantics=("parallel",)),
    )(x, W, asel, eind)

    out = pl.pallas_call(
        _main_kernel,
        grid=(B, N // R),
        in_specs=[
            pl.BlockSpec((1, R, 3), lambda b, i: (b, i, 0)),
            pl.BlockSpec((1, 3, N), lambda b, i: (b, 0, 0)),
            pl.BlockSpec((1, N, 2 * HD), lambda b, i: (b, 0, 0)),
            pl.BlockSpec((1, R, IN_F), lambda b, i: (b, i, 0)),
            pl.BlockSpec((1, HD), lambda b, i: (0, 0)),
            pl.BlockSpec((1, HD), lambda b, i: (0, 0)),
        ],
        out_specs=pl.BlockSpec((1, R, IN_F), lambda b, i: (b, i, 0)),
        out_shape=jax.ShapeDtypeStruct((B, N, IN_F), jnp.float32),
        compiler_params=pltpu.CompilerParams(
            dimension_semantics=("parallel", "parallel")),
    )(positions, posT, hwe, x, lns, lnb)
    return out


# speedup vs baseline: 30.5583x; 1.1015x over previous
import jax
import jax.numpy as jnp
from jax.experimental import pallas as pl
from jax.experimental.pallas import tpu as pltpu

B, N, IN_F = 2, 4096, 128
H, D = 4, 32
K = 32
R = 512


def _prologue_kernel(x_ref, w_ref, asel_ref, eind_ref, hwe_ref):
    x = x_ref[0]
    W = w_ref[...]
    h = jnp.dot(x.astype(jnp.bfloat16), W.astype(jnp.bfloat16),
                preferred_element_type=jnp.float32)
    e = jnp.dot(h, asel_ref[...], precision=jax.lax.Precision.HIGHEST)
    f = jnp.where(e >= 0.0, e, 0.2 * e)
    fmax = jnp.max(f, axis=0, keepdims=True)
    expf = jnp.exp(f - fmax)
    expfb = jnp.dot(expf, eind_ref[...],
                    precision=jax.lax.Precision.HIGHEST)
    hwe_ref[0, :, : H * D] = h * expfb
    hwe_ref[0, :, H * D:] = expfb


def _main_kernel(pos_ref, post_ref, hwe_ref, x_ref, lns_ref, lnb_ref, out_ref):
    pos = pos_ref[0]
    posT = post_ref[0]
    dot = jnp.dot(pos.astype(jnp.bfloat16), posT.astype(jnp.bfloat16),
                  preferred_element_type=jnp.float32)
    p2b = jnp.sum(pos * pos, axis=1, keepdims=True)
    p2f = jnp.sum(posT * posT, axis=0, keepdims=True)
    d2 = p2b + p2f - 2.0 * dot

    def body(_, m):
        return jnp.min(jnp.where(d2 > m, d2, jnp.inf), axis=1, keepdims=True)

    m0 = jnp.full((R, 1), -jnp.inf, dtype=jnp.float32)
    tau = jax.lax.fori_loop(0, K, body, m0)
    A = (d2 <= tau).astype(jnp.float32)

    nm = jnp.dot(A, hwe_ref[0], precision=jax.lax.Precision.HIGHEST)
    hp = nm[:, : H * D] / nm[:, H * D:]
    y = hp + x_ref[0]
    mu = jnp.mean(y, axis=1, keepdims=True)
    yc = y - mu
    var = jnp.mean(yc * yc, axis=1, keepdims=True)
    out = yc * jax.lax.rsqrt(var + 1e-5) * lns_ref[...] + lnb_ref[...]
    out_ref[0] = out


def kernel(x, positions, W, a_src, a_dst, ln_scale, ln_bias, topk):
    HD = H * D
    a_flat = (a_src + a_dst).reshape(HD)
    grp = jnp.arange(HD, dtype=jnp.int32) // D
    heads = jnp.arange(H, dtype=jnp.int32)
    asel = jnp.where(grp[:, None] == heads[None, :], a_flat[:, None], 0.0)
    eind = (grp[None, :] == heads[:, None]).astype(jnp.float32)
    posT = jnp.transpose(positions, (0, 2, 1))
    lns = ln_scale.reshape(1, HD)
    lnb = ln_bias.reshape(1, HD)

    hwe = pl.pallas_call(
        _prologue_kernel,
        grid=(B,),
        in_specs=[
            pl.BlockSpec((1, N, IN_F), lambda b: (b, 0, 0)),
            pl.BlockSpec((IN_F, HD), lambda b: (0, 0)),
            pl.BlockSpec((HD, H), lambda b: (0, 0)),
            pl.BlockSpec((H, HD), lambda b: (0, 0)),
        ],
        out_specs=pl.BlockSpec((1, N, 2 * HD), lambda b: (b, 0, 0)),
        out_shape=jax.ShapeDtypeStruct((B, N, 2 * HD), jnp.float32),
        compiler_params=pltpu.CompilerParams(
            dimension_semantics=("parallel",)),
    )(x, W, asel, eind)

    out = pl.pallas_call(
        _main_kernel,
        grid=(B, N // R),
        in_specs=[
            pl.BlockSpec((1, R, 3), lambda b, i: (b, i, 0)),
            pl.BlockSpec((1, 3, N), lambda b, i: (b, 0, 0)),
            pl.BlockSpec((1, N, 2 * HD), lambda b, i: (b, 0, 0)),
            pl.BlockSpec((1, R, IN_F), lambda b, i: (b, i, 0)),
            pl.BlockSpec((1, HD), lambda b, i: (0, 0)),
            pl.BlockSpec((1, HD), lambda b, i: (0, 0)),
        ],
        out_specs=pl.BlockSpec((1, R, IN_F), lambda b, i: (b, i, 0)),
        out_shape=jax.ShapeDtypeStruct((B, N, IN_F), jnp.float32),
        compiler_params=pltpu.CompilerParams(
            dimension_semantics=("parallel", "parallel")),
    )(positions, posT, hwe, x, lns, lnb)
    return out
